# baseline (speedup 1.0000x reference)
import functools

import jax
import jax.numpy as jnp
from jax import lax
from jax.experimental import pallas as pl
from jax.experimental.pallas import tpu as pltpu
from jax.experimental.pallas import tpu_sc as plsc

BLK = 4000
CHUNK = 128


def kernel(color_idx, table):
    (B,) = color_idx.shape
    V, D = table.shape
    info = plsc.get_sparse_core_info()
    NC, NS = info.num_cores, info.num_subcores
    NW = NC * NS
    L = info.num_lanes
    b_per_w = B // NW
    nch = b_per_w // CHUNK

    idx1 = color_idx.astype(jnp.int32)

    nblk = (V // 2) // BLK

    def pack_body(x1_ref, x2_ref, o_ref):
        o_ref[...] = jnp.concatenate([x1_ref[...], x2_ref[...]], axis=1)

    compact = pl.pallas_call(
        pack_body,
        grid=(nblk,),
        in_specs=[
            pl.BlockSpec((BLK, D), lambda i: (i, 0)),
            pl.BlockSpec((BLK, D), lambda i: (i + nblk, 0)),
        ],
        out_specs=pl.BlockSpec((BLK, 2 * D), lambda i: (i, 0)),
        out_shape=jax.ShapeDtypeStruct((V // 2, 2 * D), jnp.float32),
    )(table, table)

    mesh = plsc.VectorSubcoreMesh(core_axis_name="c", subcore_axis_name="s")

    @functools.partial(
        pl.kernel,
        mesh=mesh,
        out_type=jax.ShapeDtypeStruct((B, D), jnp.float32),
        scratch_types=[
            pltpu.VMEM((b_per_w,), jnp.int32),
            pltpu.VMEM((nch, CHUNK), jnp.int32),
            pltpu.VMEM((CHUNK, 2 * D), jnp.float32),
            pltpu.VMEM((CHUNK, 2 * D), jnp.float32),
            pltpu.VMEM((b_per_w, D), jnp.float32),
            pltpu.SemaphoreType.DMA,
            pltpu.SemaphoreType.DMA,
        ],
        compiler_params=pltpu.CompilerParams(
            use_tc_tiling_on_sc=True, needs_layout_passes=False
        ),
    )
    def gather(
        idx_hbm, compact_hbm, out_hbm,
        idx_v, pidx_v, pairs_a, pairs_b, rows_v, sem_a, sem_b,
    ):
        wid = lax.axis_index("s") * NC + lax.axis_index("c")
        base = wid * b_per_w
        pltpu.sync_copy(idx_hbm.at[pl.ds(base, b_per_w)], idx_v)
        V2 = V // 2
        for j in range(nch):
            for g in range(CHUNK // L):
                iv = idx_v[pl.ds(j * CHUNK + g * L, L)]
                hi = jnp.where(iv >= V2, jnp.int32(V2), jnp.int32(0))
                pidx_v[j, pl.ds(g * L, L)] = iv - hi
        bufs = (pairs_a, pairs_b)
        sems = (sem_a, sem_b)

        def issue(j):
            pltpu.async_copy(
                compact_hbm.at[pidx_v.at[j]], bufs[j % 2], sems[j % 2]
            )

        issue(0)
        for j in range(nch):
            if j + 1 < nch:
                issue(j + 1)
            pltpu.make_async_copy(
                compact_hbm.at[pidx_v.at[j]], bufs[j % 2], sems[j % 2]
            ).wait()
            pv = bufs[j % 2]
            for g in range(CHUNK // L):
                iv = idx_v[pl.ds(j * CHUNK + g * L, L)]
                rowi = lax.iota(jnp.int32, L) + g * L
                orow = rowi + j * CHUNK
                colb = jnp.where(iv >= V2, jnp.int32(D), jnp.int32(0))
                zero = jnp.zeros((L,), jnp.int32)

                def body(e, carry, pv=pv, rowi=rowi, orow=orow, colb=colb, zero=zero):
                    v = plsc.load_gather(pv, [rowi, colb + e])
                    plsc.store_scatter(rows_v, [orow, zero + e], v)
                    return carry

                lax.fori_loop(0, D, body, 0)
        pltpu.sync_copy(rows_v, out_hbm.at[pl.ds(base, b_per_w)])

    return gather(idx1, compact)
